# baseline (speedup 1.0000x reference)
import functools

import jax
import jax.numpy as jnp
from jax import lax
from jax.experimental import pallas as pl
from jax.experimental.pallas import tpu as pltpu
from jax.experimental.pallas import tpu_sc as plsc

_VOCAB = 1000000
_D = 64
_B = 16384
_LANES = 16

_NC = 2
_NS = 16
_NW = _NC * _NS

_CH = 128
_NCHUNK = _B // (_NW * _CH)
_G = _NW * _NCHUNK


def _make_sc_call():
    mesh = plsc.VectorSubcoreMesh(core_axis_name="c", subcore_axis_name="s")

    @functools.partial(
        pl.kernel,
        out_type=jax.ShapeDtypeStruct((_B, 3 * _D), jnp.float32),
        mesh=mesh,
        compiler_params=pltpu.CompilerParams(use_tc_tiling_on_sc=False),
        scratch_types=[
            pltpu.VMEM((_CH,), jnp.int32),
            pltpu.VMEM((4, _CH), jnp.int32),
            pltpu.VMEM((4, _CH), jnp.int32),
            pltpu.VMEM((_CH, _D), jnp.float32),
            pltpu.VMEM((4 * _CH, _D), jnp.float32),
            pltpu.VMEM((4 * _CH, _D), jnp.float32),
            pltpu.VMEM((_CH, _D), jnp.float32),
            pltpu.VMEM((_CH, _D), jnp.float32),
            pltpu.SemaphoreType.DMA,
        ],
    )
    def call(opc_hbm, op1_hbm, op2_hbm, table_hbm, out_hbm,
             idx0_v, idx1_v, idx2_v, rows0_v, rows1_v, rows2_v,
             acc1_v, acc2_v, sem):
        wid = lax.axis_index("s") * _NC + lax.axis_index("c")
        quarter = jnp.float32(0.25)

        for c in range(_NCHUNK):
            g = wid * _NCHUNK + c
            pltpu.sync_copy(opc_hbm.at[g], idx0_v)
            pltpu.sync_copy(op1_hbm.at[g], idx1_v)
            pltpu.sync_copy(op2_hbm.at[g], idx2_v)
            copies = [pltpu.async_copy(table_hbm.at[idx0_v], rows0_v, sem)]
            for q in range(4):
                copies.append(pltpu.async_copy(
                    table_hbm.at[idx1_v.at[q]],
                    rows1_v.at[pl.ds(q * _CH, _CH)], sem))
            for q in range(4):
                copies.append(pltpu.async_copy(
                    table_hbm.at[idx2_v.at[q]],
                    rows2_v.at[pl.ds(q * _CH, _CH)], sem))
            for cp in copies:
                cp.wait()
            pltpu.sync_copy(
                rows0_v, out_hbm.at[pl.ds(g * _CH, _CH), pl.ds(0, _D)]
            )

            def body(i, _):
                base = 4 * i
                for k in range(_D // _LANES):
                    s = pl.ds(k * _LANES, _LANES)
                    a1 = (
                        rows1_v[base, s] + rows1_v[base + 1, s]
                        + rows1_v[base + 2, s] + rows1_v[base + 3, s]
                    ) * quarter
                    acc1_v[i, s] = a1
                    a2 = (
                        rows2_v[base, s] + rows2_v[base + 1, s]
                        + rows2_v[base + 2, s] + rows2_v[base + 3, s]
                    ) * quarter
                    acc2_v[i, s] = a2
                return 0

            lax.fori_loop(0, _CH, body, 0, unroll=False)

            pltpu.sync_copy(
                acc1_v, out_hbm.at[pl.ds(g * _CH, _CH), pl.ds(_D, _D)]
            )
            pltpu.sync_copy(
                acc2_v, out_hbm.at[pl.ds(g * _CH, _CH), pl.ds(2 * _D, _D)]
            )

    return call


_sc_call = _make_sc_call()


@jax.jit
def kernel(opcode_idx, op1_idx, op2_idx, table):
    opc = opcode_idx.astype(jnp.int32).reshape(_G, _CH)
    op1 = op1_idx.astype(jnp.int32).reshape(_G, 4, _CH)
    op2 = op2_idx.astype(jnp.int32).reshape(_G, 4, _CH)
    return _sc_call(opc, op1, op2, table)
